# baseline (speedup 1.0000x reference)
import functools

import jax
import jax.numpy as jnp
from jax import lax
from jax.experimental import pallas as pl
from jax.experimental.pallas import tpu as pltpu
from jax.experimental.pallas import tpu_sc as plsc

_VOCAB = 10000
_EMBED = 128
_BATCH = 16384

_NC = 2
_NS = 16
_NW = _NC * _NS
_CHUNK = 128
_NCH = _BATCH // (_NW * _CHUNK)
_BPW = _NCH * _CHUNK

_TC_GRID = 10
_ROWS = 1024
_VPAD = _TC_GRID * _ROWS


def _tc_precompute_body(table_ref, w_ref, b_ref, y_ref):
    i = pl.program_id(0)
    t = jnp.maximum(table_ref[...], 0.0)
    acc = lax.dot_general(
        t, w_ref[...], (((1,), (1,)), ((), ())),
        preferred_element_type=jnp.float32)
    yv = jnp.transpose(acc)
    y_ref[:, pl.ds(i * _ROWS, _ROWS)] = jax.nn.sigmoid(yv + b_ref[0, 0])


def _tc_precompute(table, w, b):
    table = pltpu.with_memory_space_constraint(table, pltpu.MemorySpace.HBM)
    return pl.pallas_call(
        _tc_precompute_body,
        grid=(_TC_GRID,),
        in_specs=[
            pl.BlockSpec((_ROWS, _EMBED), lambda i: (i, 0)),
            pl.BlockSpec((1, _EMBED), lambda i: (0, 0)),
            pl.BlockSpec((1, 1), lambda i: (0, 0)),
        ],
        out_specs=pl.BlockSpec((1, _VPAD), lambda i: (0, 0)),
        out_shape=jax.ShapeDtypeStruct((1, _VPAD), jnp.float32),
    )(table, w.reshape(1, _EMBED), b)


_sc_mesh = plsc.VectorSubcoreMesh(
    core_axis_name="c", subcore_axis_name="s", num_cores=_NC
)


@functools.partial(
    pl.kernel,
    mesh=_sc_mesh,
    out_type=jax.ShapeDtypeStruct((_BATCH,), jnp.float32),
    scratch_types=[
        pltpu.VMEM((_BPW,), jnp.int32),
        pltpu.VMEM((_BPW,), jnp.float32),
        pltpu.SemaphoreType.DMA,
    ],
)
def _sc_gather(idx_hbm, y_hbm, out_hbm, idx_v, vals_v, sem):
    wid = lax.axis_index("s") * _NC + lax.axis_index("c")
    base = wid * _BPW
    pltpu.sync_copy(idx_hbm.at[pl.ds(base, _BPW)], idx_v)
    copies = [
        pltpu.async_copy(
            y_hbm.at[0].at[idx_v.at[pl.ds(j * _CHUNK, _CHUNK)]],
            vals_v.at[pl.ds(j * _CHUNK, _CHUNK)],
            sem,
        )
        for j in range(_NCH)
    ]
    for c in copies:
        c.wait()
    pltpu.sync_copy(vals_v, out_hbm.at[pl.ds(base, _BPW)])


def kernel(x, table, kernel, bias):
    y = _tc_precompute(table, kernel, bias)
    return _sc_gather(x.astype(jnp.int32), y).reshape(_BATCH, 1)


# speedup vs baseline: 1.1069x; 1.1069x over previous
import functools

import jax
import jax.numpy as jnp
from jax import lax
from jax.experimental import pallas as pl
from jax.experimental.pallas import tpu as pltpu
from jax.experimental.pallas import tpu_sc as plsc

_VOCAB = 10000
_EMBED = 128
_BATCH = 16384

_NC = 2
_NS = 16
_NW = _NC * _NS
_CHUNK = 128
_NCH = _BATCH // (_NW * _CHUNK)
_BPW = _NCH * _CHUNK

_TC_CHUNKS = [(i * 1024, 1024) for i in range(9)] + [(9216, 784)]


def _tc_precompute_body(table_hbm, w_ref, b_ref, y_ref, tbuf, sems):
    copies = []
    for k, (off, sz) in enumerate(_TC_CHUNKS):
        copies.append(pltpu.make_async_copy(
            table_hbm.at[pl.ds(off, sz), :],
            tbuf.at[pl.ds(off, sz), :],
            sems.at[k],
        ))
        copies[k].start()
    w = w_ref[...]
    b = b_ref[0, 0]
    for k, (off, sz) in enumerate(_TC_CHUNKS):
        copies[k].wait()
        t = jnp.maximum(tbuf[pl.ds(off, sz), :], 0.0)
        acc = lax.dot_general(
            t, w, (((1,), (1,)), ((), ())),
            preferred_element_type=jnp.float32)
        yv = jnp.transpose(acc)
        y_ref[:, pl.ds(off, sz)] = jax.nn.sigmoid(yv + b)


def _tc_precompute(table, w, b):
    table = pltpu.with_memory_space_constraint(table, pltpu.MemorySpace.HBM)
    return pl.pallas_call(
        _tc_precompute_body,
        in_specs=[
            pl.BlockSpec(memory_space=pl.ANY),
            pl.BlockSpec((1, _EMBED), lambda: (0, 0)),
            pl.BlockSpec((1, 1), lambda: (0, 0)),
        ],
        out_shape=jax.ShapeDtypeStruct((1, _VOCAB), jnp.float32),
        scratch_shapes=[
            pltpu.VMEM((_VOCAB, _EMBED), jnp.float32),
            pltpu.SemaphoreType.DMA((len(_TC_CHUNKS),)),
        ],
    )(table, w.reshape(1, _EMBED), b)


_sc_mesh = plsc.VectorSubcoreMesh(
    core_axis_name="c", subcore_axis_name="s", num_cores=_NC
)


@functools.partial(
    pl.kernel,
    mesh=_sc_mesh,
    out_type=jax.ShapeDtypeStruct((_BATCH,), jnp.float32),
    scratch_types=[
        pltpu.VMEM((_BPW,), jnp.int32),
        pltpu.VMEM((_BPW,), jnp.float32),
        pltpu.VMEM_SHARED((_VOCAB,), jnp.float32),
        pltpu.SemaphoreType.DMA,
    ],
)
def _sc_gather(idx_hbm, y_hbm, out_hbm, idx_v, vals_v, y_sh, sem):
    sid = lax.axis_index("s")
    wid = sid * _NC + lax.axis_index("c")
    base = wid * _BPW
    @pl.when(sid == 0)
    def _():
        pltpu.sync_copy(y_hbm.at[0], y_sh)
    pltpu.sync_copy(idx_hbm.at[pl.ds(base, _BPW)], idx_v)
    plsc.subcore_barrier()
    copies = [
        pltpu.async_copy(
            y_sh.at[idx_v.at[pl.ds(j * _CHUNK, _CHUNK)]],
            vals_v.at[pl.ds(j * _CHUNK, _CHUNK)],
            sem,
        )
        for j in range(_NCH)
    ]
    for c in copies:
        c.wait()
    pltpu.sync_copy(vals_v, out_hbm.at[pl.ds(base, _BPW)])


def kernel(x, table, kernel, bias):
    y = _tc_precompute(table, kernel, bias)
    return _sc_gather(x.astype(jnp.int32), y).reshape(_BATCH, 1)


# speedup vs baseline: 1.1764x; 1.0628x over previous
import functools

import jax
import jax.numpy as jnp
from jax import lax
from jax.experimental import pallas as pl
from jax.experimental.pallas import tpu as pltpu
from jax.experimental.pallas import tpu_sc as plsc

_VOCAB = 10000
_EMBED = 128
_BATCH = 16384

_NC = 1
_NS = 16
_NW = _NC * _NS
_CHUNK = 128
_NCH = _BATCH // (_NW * _CHUNK)
_BPW = _NCH * _CHUNK

_TC_CHUNKS = [(i * 1024, 1024) for i in range(9)] + [(9216, 784)]


def _tc_precompute_body(table_hbm, w_ref, b_ref, y_ref, tbuf, sems):
    copies = []
    for k, (off, sz) in enumerate(_TC_CHUNKS):
        copies.append(pltpu.make_async_copy(
            table_hbm.at[pl.ds(off, sz), :],
            tbuf.at[pl.ds(off, sz), :],
            sems.at[k],
        ))
        copies[k].start()
    w = w_ref[...]
    b = b_ref[0, 0]
    for k, (off, sz) in enumerate(_TC_CHUNKS):
        copies[k].wait()
        t = jnp.maximum(tbuf[pl.ds(off, sz), :], 0.0)
        acc = lax.dot_general(
            t, w, (((1,), (1,)), ((), ())),
            preferred_element_type=jnp.float32)
        yv = jnp.transpose(acc)
        y_ref[:, pl.ds(off, sz)] = jax.nn.sigmoid(yv + b)


def _tc_precompute(table, w, b):
    table = pltpu.with_memory_space_constraint(table, pltpu.MemorySpace.HBM)
    return pl.pallas_call(
        _tc_precompute_body,
        in_specs=[
            pl.BlockSpec(memory_space=pl.ANY),
            pl.BlockSpec((1, _EMBED), lambda: (0, 0)),
            pl.BlockSpec((1, 1), lambda: (0, 0)),
        ],
        out_shape=jax.ShapeDtypeStruct((1, _VOCAB), jnp.float32),
        scratch_shapes=[
            pltpu.VMEM((_VOCAB, _EMBED), jnp.float32),
            pltpu.SemaphoreType.DMA((len(_TC_CHUNKS),)),
        ],
    )(table, w.reshape(1, _EMBED), b)


_sc_mesh = plsc.VectorSubcoreMesh(
    core_axis_name="c", subcore_axis_name="s", num_cores=_NC
)


@functools.partial(
    pl.kernel,
    mesh=_sc_mesh,
    out_type=jax.ShapeDtypeStruct((_BATCH,), jnp.float32),
    scratch_types=[
        pltpu.VMEM((_BPW,), jnp.int32),
        pltpu.VMEM((_BPW,), jnp.float32),
        pltpu.VMEM_SHARED((_VOCAB,), jnp.float32),
        pltpu.SemaphoreType.DMA,
    ],
)
def _sc_gather(idx_hbm, y_hbm, out_hbm, idx_v, vals_v, y_sh, sem):
    sid = lax.axis_index("s")
    wid = sid * _NC + lax.axis_index("c")
    base = wid * _BPW
    @pl.when(sid == 0)
    def _():
        pltpu.sync_copy(y_hbm.at[0], y_sh)
    pltpu.sync_copy(idx_hbm.at[pl.ds(base, _BPW)], idx_v)
    plsc.subcore_barrier()
    copies = [
        pltpu.async_copy(
            y_sh.at[idx_v.at[pl.ds(j * _CHUNK, _CHUNK)]],
            vals_v.at[pl.ds(j * _CHUNK, _CHUNK)],
            sem,
        )
        for j in range(_NCH)
    ]
    for c in copies:
        c.wait()
    pltpu.sync_copy(vals_v, out_hbm.at[pl.ds(base, _BPW)])


def kernel(x, table, kernel, bias):
    y = _tc_precompute(table, kernel, bias)
    return _sc_gather(x.astype(jnp.int32), y).reshape(_BATCH, 1)
